# Initial kernel scaffold; baseline (speedup 1.0000x reference)
import jax
import jax.numpy as jnp
from jax.experimental import pallas as pl


def kernel(x, table):
    raise NotImplementedError("write your pallas kernel here")


# speedup vs baseline: 1.5356x; 1.5356x over previous
import functools

import jax
import jax.numpy as jnp
from jax import lax
from jax.experimental import pallas as pl
from jax.experimental.pallas import tpu as pltpu
from jax.experimental.pallas import tpu_sc as plsc

_NUM_WORKERS = 32
_CHUNK = 32


def _emb_kernel_body(n_chunks, rows_per_worker, x_hbm, table_hbm, out_hbm,
                     idx_v, rows0, rows1, sem0, sem1):
    wid = lax.axis_index("s") * 2 + lax.axis_index("c")
    base = wid * rows_per_worker
    pltpu.sync_copy(x_hbm.at[wid], idx_v)

    bufs = (rows0, rows1)
    sems = (sem0, sem1)
    handles = [None, None]
    handles[0] = pltpu.async_copy(table_hbm.at[idx_v.at[0]], bufs[0], sems[0])
    for c in range(n_chunks):
        nxt = c + 1
        if nxt < n_chunks:
            handles[nxt % 2] = pltpu.async_copy(
                table_hbm.at[idx_v.at[nxt]], bufs[nxt % 2], sems[nxt % 2])
        handles[c % 2].wait()
        pltpu.sync_copy(bufs[c % 2], out_hbm.at[pl.ds(base + c * _CHUNK, _CHUNK)])


def kernel(x, table):
    batch, seq = x.shape
    _, d_model = table.shape
    n = batch * seq
    rows_per_worker = n // _NUM_WORKERS
    n_chunks = rows_per_worker // _CHUNK
    x_flat = x.reshape(_NUM_WORKERS, n_chunks, _CHUNK).astype(jnp.int32)

    mesh = plsc.VectorSubcoreMesh(core_axis_name="c", subcore_axis_name="s")
    emb = pl.kernel(
        functools.partial(_emb_kernel_body, n_chunks, rows_per_worker),
        mesh=mesh,
        out_type=jax.ShapeDtypeStruct((n, d_model), jnp.float32),
        scratch_types=[
            pltpu.VMEM((n_chunks, _CHUNK), jnp.int32),
            pltpu.VMEM((_CHUNK, d_model), jnp.float32),
            pltpu.VMEM((_CHUNK, d_model), jnp.float32),
            pltpu.SemaphoreType.DMA,
            pltpu.SemaphoreType.DMA,
        ],
    )
    out = emb(x_flat, table)
    return out.reshape(batch, seq, d_model)
